# baseline (speedup 1.0000x reference)
import functools

import jax
import jax.numpy as jnp
from jax import lax
from jax.experimental import pallas as pl
from jax.experimental.pallas import tpu as pltpu
from jax.experimental.pallas import tpu_sc as plsc

B, S, H, V = 4, 8192, 128, 100
BS = B * S
VP = 128
CHUNK = 128


def _tables_body(emb_ref, w1t_ref, b1_ref, w2t_ref, b2_ref, t1_ref, t2_ref):
    t1 = jnp.dot(emb_ref[:], w1t_ref[:],
                 preferred_element_type=jnp.float32) + b1_ref[:]
    t1_ref[:] = t1
    t2_ref[:] = jnp.dot(t1, w2t_ref[:],
                        preferred_element_type=jnp.float32) + b2_ref[:]


def _compute_tables(emb_pad, w1t, b1r, w2t, b2r):
    return pl.pallas_call(
        _tables_body,
        out_shape=[jax.ShapeDtypeStruct((VP, H), jnp.float32)] * 2,
    )(emb_pad, w1t, b1r, w2t, b2r)


def _make_sc_gather(nc, ns):
    nw = nc * ns
    tok_per_w = BS // nw
    nchunk = tok_per_w // CHUNK
    mesh = plsc.VectorSubcoreMesh(core_axis_name="c", subcore_axis_name="s")

    @functools.partial(
        pl.kernel,
        out_type=[jax.ShapeDtypeStruct((BS, H), jnp.float32)] * 3,
        mesh=mesh,
        scratch_types=[
            pltpu.VMEM((nchunk, CHUNK), jnp.int32),
            pltpu.VMEM((CHUNK, H), jnp.float32),
            pltpu.SemaphoreType.DMA,
        ],
    )
    def sc_gather(ids_hbm, t0, t1, t2, y0, y1, y2, idx_v, buf, sem):
        wid = lax.axis_index("s") * nc + lax.axis_index("c")
        base = wid * tok_per_w
        pltpu.sync_copy(ids_hbm.at[pl.ds(wid * nchunk, nchunk)], idx_v)
        for tab, out in ((t0, y0), (t1, y1), (t2, y2)):
            for c in range(nchunk):
                pltpu.async_copy(tab.at[idx_v.at[c]], buf, sem).wait()
                pltpu.sync_copy(buf, out.at[pl.ds(base + c * CHUNK, CHUNK)])

    return sc_gather


def kernel(input_ids, emb, W1, b1, W2, b2):
    info = plsc.get_sparse_core_info()
    emb_pad = jnp.zeros((VP, H), jnp.float32).at[:V].set(emb)
    t1, t2 = _compute_tables(emb_pad, W1.T, b1.reshape(1, H),
                             W2.T, b2.reshape(1, H))
    ids2 = input_ids.reshape(BS // CHUNK, CHUNK)
    sc_gather = _make_sc_gather(info.num_cores, info.num_subcores)
    y0, y1, y2 = sc_gather(ids2, emb_pad, t1, t2)
    return (y0.reshape(B, S, H), y1.reshape(B, S, H), y2.reshape(B, S, H))


# speedup vs baseline: 1.0863x; 1.0863x over previous
import functools

import jax
import jax.numpy as jnp
from jax import lax
from jax.experimental import pallas as pl
from jax.experimental.pallas import tpu as pltpu
from jax.experimental.pallas import tpu_sc as plsc

B, S, H, V = 4, 8192, 128, 100
BS = B * S
VP = 128
CHUNK = 128


def _tables_body(emb_ref, w1t_ref, b1_ref, w2t_ref, b2_ref, t1_ref, t2_ref):
    t1 = jnp.dot(emb_ref[:], w1t_ref[:],
                 preferred_element_type=jnp.float32) + b1_ref[:]
    t1_ref[:] = t1
    t2_ref[:] = jnp.dot(t1, w2t_ref[:],
                        preferred_element_type=jnp.float32) + b2_ref[:]


def _compute_tables(emb_pad, w1t, b1r, w2t, b2r):
    return pl.pallas_call(
        _tables_body,
        out_shape=[jax.ShapeDtypeStruct((VP, H), jnp.float32)] * 2,
    )(emb_pad, w1t, b1r, w2t, b2r)


NBUF = 4


def _make_sc_gather(nc, ns):
    nw = nc * ns
    tok_per_w = BS // nw
    nchunk = tok_per_w // CHUNK
    mesh = plsc.VectorSubcoreMesh(core_axis_name="c", subcore_axis_name="s")

    @functools.partial(
        pl.kernel,
        out_type=[jax.ShapeDtypeStruct((BS, H), jnp.float32)] * 3,
        mesh=mesh,
        scratch_types=[
            pltpu.VMEM((nchunk, CHUNK), jnp.int32),
            pltpu.VMEM((NBUF, CHUNK, H), jnp.float32),
            pltpu.SemaphoreType.DMA,
            pltpu.SemaphoreType.DMA,
        ],
    )
    def sc_gather(ids_hbm, t0, t1, t2, y0, y1, y2, idx_v, buf, gsem, wsem):
        wid = lax.axis_index("s") * nc + lax.axis_index("c")
        base = wid * tok_per_w
        pltpu.sync_copy(ids_hbm.at[pl.ds(wid * nchunk, nchunk)], idx_v)
        tasks = [(tab, out, c)
                 for tab, out in ((t0, y0), (t1, y1), (t2, y2))
                 for c in range(nchunk)]

        def gather(i):
            tab, _, c = tasks[i]
            return pltpu.async_copy(tab.at[idx_v.at[c]], buf.at[i % NBUF],
                                    gsem)

        def write(i):
            _, out, c = tasks[i]
            return pltpu.async_copy(buf.at[i % NBUF],
                                    out.at[pl.ds(base + c * CHUNK, CHUNK)],
                                    wsem)

        gds = [gather(i) for i in range(NBUF - 1)]
        wds = []
        for i in range(len(tasks)):
            if i + NBUF - 1 < len(tasks):
                if i >= 1:
                    wds[i - 1].wait()
                gds.append(gather(i + NBUF - 1))
            elif i >= 1:
                wds[i - 1].wait()
            gds[i].wait()
            wds.append(write(i))
        wds[-1].wait()

    return sc_gather


def kernel(input_ids, emb, W1, b1, W2, b2):
    info = plsc.get_sparse_core_info()
    emb_pad = jnp.zeros((VP, H), jnp.float32).at[:V].set(emb)
    t1, t2 = _compute_tables(emb_pad, W1.T, b1.reshape(1, H),
                             W2.T, b2.reshape(1, H))
    ids2 = input_ids.reshape(BS // CHUNK, CHUNK)
    sc_gather = _make_sc_gather(info.num_cores, info.num_subcores)
    y0, y1, y2 = sc_gather(ids2, emb_pad, t1, t2)
    return (y0.reshape(B, S, H), y1.reshape(B, S, H), y2.reshape(B, S, H))


# speedup vs baseline: 3.1421x; 2.8924x over previous
import functools

import jax
import jax.numpy as jnp
from jax import lax
from jax.experimental import pallas as pl
from jax.experimental.pallas import tpu as pltpu
from jax.experimental.pallas import tpu_sc as plsc

B, S, H, V = 4, 8192, 128, 100
BS = B * S
VP = 128
CHUNK = 128


def _tables_body(emb_ref, w1t_ref, b1_ref, w2t_ref, b2_ref, t1_ref, t2_ref):
    t1 = jnp.dot(emb_ref[:], w1t_ref[:],
                 preferred_element_type=jnp.float32) + b1_ref[:]
    t1_ref[:] = t1
    t2_ref[:] = jnp.dot(t1, w2t_ref[:],
                        preferred_element_type=jnp.float32) + b2_ref[:]


def _compute_tables(emb_pad, w1t, b1r, w2t, b2r):
    return pl.pallas_call(
        _tables_body,
        out_shape=[jax.ShapeDtypeStruct((VP, H), jnp.float32)] * 2,
    )(emb_pad, w1t, b1r, w2t, b2r)


NBUF = 4


def _make_sc_gather(nc, ns):
    nw = nc * ns
    tok_per_w = BS // nw
    nchunk = tok_per_w // CHUNK
    mesh = plsc.VectorSubcoreMesh(core_axis_name="c", subcore_axis_name="s")

    @functools.partial(
        pl.kernel,
        out_type=[jax.ShapeDtypeStruct((BS, H), jnp.float32)] * 3,
        mesh=mesh,
        scratch_types=[
            pltpu.VMEM((nchunk, CHUNK), jnp.int32),
            pltpu.VMEM((NBUF, CHUNK, H), jnp.float32),
            pltpu.VMEM_SHARED((VP, H), jnp.float32),
            pltpu.VMEM_SHARED((VP, H), jnp.float32),
            pltpu.VMEM_SHARED((VP, H), jnp.float32),
            pltpu.SemaphoreType.DMA,
            pltpu.SemaphoreType.DMA,
        ],
    )
    def sc_gather(ids_hbm, t0, t1, t2, y0, y1, y2,
                  idx_v, buf, tv0, tv1, tv2, gsem, wsem):
        wid = lax.axis_index("s") * nc + lax.axis_index("c")
        base = wid * tok_per_w
        pltpu.sync_copy(ids_hbm.at[pl.ds(wid * nchunk, nchunk)], idx_v)

        @pl.when(lax.axis_index("s") == 0)
        def _stage_tables():
            pltpu.sync_copy(t0, tv0)
            pltpu.sync_copy(t1, tv1)
            pltpu.sync_copy(t2, tv2)

        plsc.subcore_barrier()
        tasks = [(tab, out, c)
                 for tab, out in ((tv0, y0), (tv1, y1), (tv2, y2))
                 for c in range(nchunk)]

        def gather(i):
            tab, _, c = tasks[i]
            return pltpu.async_copy(tab.at[idx_v.at[c]], buf.at[i % NBUF],
                                    gsem)

        def write(i):
            _, out, c = tasks[i]
            return pltpu.async_copy(buf.at[i % NBUF],
                                    out.at[pl.ds(base + c * CHUNK, CHUNK)],
                                    wsem)

        gds = [gather(i) for i in range(NBUF - 1)]
        wds = []
        for i in range(len(tasks)):
            if i + NBUF - 1 < len(tasks):
                if i >= 1:
                    wds[i - 1].wait()
                gds.append(gather(i + NBUF - 1))
            elif i >= 1:
                wds[i - 1].wait()
            gds[i].wait()
            wds.append(write(i))
        wds[-1].wait()

    return sc_gather


def kernel(input_ids, emb, W1, b1, W2, b2):
    info = plsc.get_sparse_core_info()
    emb_pad = jnp.zeros((VP, H), jnp.float32).at[:V].set(emb)
    t1, t2 = _compute_tables(emb_pad, W1.T, b1.reshape(1, H),
                             W2.T, b2.reshape(1, H))
    ids2 = input_ids.reshape(BS // CHUNK, CHUNK)
    sc_gather = _make_sc_gather(info.num_cores, info.num_subcores)
    y0, y1, y2 = sc_gather(ids2, emb_pad, t1, t2)
    return (y0.reshape(B, S, H), y1.reshape(B, S, H), y2.reshape(B, S, H))
